# baseline (speedup 1.0000x reference)
import functools

import jax
import jax.numpy as jnp
from jax import lax
from jax.experimental import pallas as pl
from jax.experimental.pallas import tpu as pltpu
from jax.experimental.pallas import tpu_sc as plsc

B, C, H, W = 8, 384, 224, 224
P = 200
NPTS = 2 * B * P
CHUNK = 104
SUB_A = 56
SUB_B = CHUNK - SUB_A
LANES = 16


def _body(tbl_hbm, cpt_hbm, npt_hbm, out_hbm, cpts_v, npts_v, idx_v,
          rows_a, sem_a):
    wid = lax.axis_index("s") * 2 + lax.axis_index("c")
    seg = wid // 2
    arr = seg // 8
    b = seg % 8
    off = (wid % 2) * 96
    use_n = arr == 1

    pltpu.sync_copy(cpt_hbm.at[b], cpts_v)
    pltpu.sync_copy(npt_hbm.at[b], npts_v)

    for s in list(range(0, P - LANES, LANES)) + [P - LANES]:
        xv = jnp.where(use_n, npts_v[0, pl.ds(s, LANES)],
                       cpts_v[0, pl.ds(s, LANES)])
        yv = jnp.where(use_n, npts_v[1, pl.ds(s, LANES)],
                       cpts_v[1, pl.ds(s, LANES)])
        idx_v[pl.ds(s, LANES)] = (b * H + yv) * W + xv

    pltpu.async_copy(tbl_hbm.at[idx_v.at[pl.ds(off, CHUNK)]], rows_a,
                     sem_a).wait()
    base = arr * (B * P) + b * P + off
    pltpu.sync_copy(rows_a, out_hbm.at[pl.ds(base, CHUNK)])


@jax.jit
def _crop_gather(tbl, cpt, npt):
    kern = functools.partial(
        pl.kernel,
        out_type=jax.ShapeDtypeStruct((NPTS, C), jnp.float32),
        mesh=plsc.VectorSubcoreMesh(core_axis_name="c",
                                    subcore_axis_name="s"),
        scratch_types=[
            pltpu.VMEM((2, P), jnp.int32),
            pltpu.VMEM((2, P), jnp.int32),
            pltpu.VMEM((P, ), jnp.int32),
            pltpu.VMEM((CHUNK, C), jnp.float32),
            pltpu.SemaphoreType.DMA,
        ],
    )(_body)
    return kern(tbl, cpt, npt)


def kernel(imgs, batch_cpoints, batch_npoints):
    tbl = imgs.transpose(0, 2, 3, 1).reshape(B * H * W, C)
    cpt = batch_cpoints.transpose(0, 2, 1)
    npt = batch_npoints.transpose(0, 2, 1)
    batch_crop_imgs = _crop_gather(tbl, cpt, npt)
    return (batch_crop_imgs, NPTS // 2, NPTS)


# speedup vs baseline: 1.0257x; 1.0257x over previous
import functools

import jax
import jax.numpy as jnp
from jax import lax
from jax.experimental import pallas as pl
from jax.experimental.pallas import tpu as pltpu
from jax.experimental.pallas import tpu_sc as plsc

B, C, H, W = 8, 384, 224, 224
P = 200
NPTS = 2 * B * P
CHUNK = 104
LANES = 16


def _body(tbl_hbm, cpt_hbm, npt_hbm, out_hbm, pts_v, idx_v, rows_v, sem):
    wid = lax.axis_index("s") * 2 + lax.axis_index("c")
    seg = wid // 2
    arr = seg // 8
    b = seg % 8
    off = (wid % 2) * 96

    def run(pts_ref):
        pltpu.sync_copy(pts_ref.at[b], pts_v)
        for s in list(range(0, P - LANES, LANES)) + [P - LANES]:
            xv = pts_v[0, pl.ds(s, LANES)]
            yv = pts_v[1, pl.ds(s, LANES)]
            idx_v[pl.ds(s, LANES)] = (b * H + yv) * W + xv
        pltpu.async_copy(tbl_hbm.at[idx_v.at[pl.ds(off, CHUNK)]], rows_v,
                         sem).wait()
        base = arr * (B * P) + b * P + off
        pltpu.sync_copy(rows_v, out_hbm.at[pl.ds(base, CHUNK)])

    @pl.when(arr == 0)
    def _():
        run(cpt_hbm)

    @pl.when(arr == 1)
    def _():
        run(npt_hbm)


@jax.jit
def _crop_gather(tbl, cpt, npt):
    kern = functools.partial(
        pl.kernel,
        out_type=jax.ShapeDtypeStruct((NPTS, C), jnp.float32),
        mesh=plsc.VectorSubcoreMesh(core_axis_name="c",
                                    subcore_axis_name="s"),
        scratch_types=[
            pltpu.VMEM((2, P), jnp.int32),
            pltpu.VMEM((P, ), jnp.int32),
            pltpu.VMEM((CHUNK, C), jnp.float32),
            pltpu.SemaphoreType.DMA,
        ],
    )(_body)
    return kern(tbl, cpt, npt)


def kernel(imgs, batch_cpoints, batch_npoints):
    tbl = imgs.transpose(0, 2, 3, 1).reshape(B * H * W, C)
    cpt = batch_cpoints.transpose(0, 2, 1)
    npt = batch_npoints.transpose(0, 2, 1)
    batch_crop_imgs = _crop_gather(tbl, cpt, npt)
    return (batch_crop_imgs, NPTS // 2, NPTS)
